# baseline (speedup 1.0000x reference)
import functools

import jax
import jax.numpy as jnp
from jax import lax
from jax.experimental import pallas as pl
from jax.experimental.pallas import tpu as pltpu
from jax.experimental.pallas import tpu_sc as plsc

N_NODES = 10000
N_EDGES = 320000
D = 128
D1 = 16
IN_FEAT = 9
NUM_GRAPHS = 16
NUM_CONVS = 3

NC = 2
NS = 16
NW = NC * NS
CH = 128
NCH = 80
IW = 16
NIW = NCH // IW
E_PAD = NW * NCH * CH
EPT = N_EDGES // NW
PAD_PT = NCH * CH - EPT
JUNK = 256
ACC_ROWS = N_NODES + JUNK
JPT = JUNK // NS

STRIPE = 624
TAIL = N_NODES - NS * STRIPE


def _sc_segment_sum(h, idx_r, d):
    mesh = plsc.VectorSubcoreMesh(core_axis_name="c", subcore_axis_name="s")

    @functools.partial(
        pl.kernel,
        mesh=mesh,
        out_type=jax.ShapeDtypeStruct((NC, N_NODES, d), jnp.float32),
        scratch_types=[
            pltpu.VMEM((2 * IW, CH), jnp.int32),
            pltpu.VMEM((CH, d), jnp.float32),
            pltpu.VMEM((CH, d), jnp.float32),
            pltpu.VMEM_SHARED((ACC_ROWS, d), jnp.float32),
            pltpu.SemaphoreType.DMA,
            pltpu.SemaphoreType.DMA,
        ],
    )
    def k(h_hbm, idx_hbm, out_hbm, idx_v, rows_a, rows_b, acc, sem_a, sem_b):
        cid = lax.axis_index("c")
        sid = lax.axis_index("s")
        wid = cid * NS + sid
        @pl.loop(0, JPT)
        def _(r):
            @pl.loop(0, d, step=16)
            def _(c):
                rows_a[r, pl.ds(c, 16)] = jnp.zeros((16,), jnp.float32)

        pltpu.sync_copy(rows_a.at[pl.ds(0, JPT)],
                        acc.at[pl.ds(N_NODES + sid * JPT, JPT)])
        r0 = sid * STRIPE
        pltpu.sync_copy(h_hbm.at[pl.ds(r0, STRIPE)],
                        acc.at[pl.ds(r0, STRIPE)])

        @pl.when(sid == NS - 1)
        def _():
            pltpu.sync_copy(h_hbm.at[pl.ds(NS * STRIPE, TAIL)],
                            acc.at[pl.ds(NS * STRIPE, TAIL)])

        plsc.subcore_barrier()

        @pl.loop(0, NIW)
        def _(w):
            pltpu.sync_copy(idx_hbm.at[wid * NIW + w], idx_v)

            @pl.loop(0, IW)
            def _(k):
                pltpu.async_copy(h_hbm.at[idx_v.at[2 * k]], rows_a,
                                 sem_a).wait()
                pltpu.sync_copy(rows_a, acc.at[idx_v.at[2 * k + 1]], add=True)

        plsc.subcore_barrier()
        pltpu.sync_copy(acc.at[pl.ds(r0, STRIPE)],
                        out_hbm.at[cid].at[pl.ds(r0, STRIPE)])

        @pl.when(sid == NS - 1)
        def _():
            pltpu.sync_copy(acc.at[pl.ds(NS * STRIPE, TAIL)],
                            out_hbm.at[cid].at[pl.ds(NS * STRIPE, TAIL)])

    return k(h, idx_r)


def _mlp_tc(p, h, W1, b1, W2, b2, d):
    BLK = 2000
    grid = (N_NODES // BLK,)

    def body(p_ref, h_ref, W1_ref, b1_ref, W2_ref, b2_ref, o_ref):
        s = p_ref[0] + p_ref[1] - h_ref[...]
        hid = jnp.dot(s, W1_ref[...], preferred_element_type=jnp.float32)
        hid = jnp.maximum(hid + b1_ref[...], 0.0)
        o_ref[...] = (
            jnp.dot(hid, W2_ref[...], preferred_element_type=jnp.float32)
            + b2_ref[...]
        )

    return pl.pallas_call(
        body,
        grid=grid,
        in_specs=[
            pl.BlockSpec((NC, BLK, d), lambda i: (0, i, 0)),
            pl.BlockSpec((BLK, d), lambda i: (i, 0)),
            pl.BlockSpec((d, D), lambda i: (0, 0)),
            pl.BlockSpec((1, D), lambda i: (0, 0)),
            pl.BlockSpec((D, D), lambda i: (0, 0)),
            pl.BlockSpec((1, D), lambda i: (0, 0)),
        ],
        out_specs=pl.BlockSpec((BLK, D), lambda i: (i, 0)),
        out_shape=jax.ShapeDtypeStruct((N_NODES, D), jnp.float32),
    )(p, h, W1[:d], b1, W2, b2)


def _pool_post_tc(h, batch3, pW1, pb1, pW2, pb2):
    BLK = 2000
    grid = (N_NODES // BLK,)

    def body(h_ref, b_ref, W1_ref, b1_ref, W2_ref, b2_ref, o_ref, sums, cnts):
        i = pl.program_id(0)

        @pl.when(i == 0)
        def _():
            sums[...] = jnp.zeros_like(sums)
            cnts[...] = jnp.zeros_like(cnts)

        gids = lax.broadcasted_iota(jnp.int32, (NUM_GRAPHS, BLK), 0)
        onehot = (gids == b_ref[0]).astype(jnp.float32)
        sums[...] += jnp.dot(onehot, h_ref[...],
                             preferred_element_type=jnp.float32)
        cnts[...] += jnp.sum(onehot, axis=1, keepdims=True)

        @pl.when(i == grid[0] - 1)
        def _():
            s = sums[...]
            m = s / jnp.maximum(cnts[...], 1.0)
            hid = (
                jnp.dot(s, W1_ref[0:D, :], preferred_element_type=jnp.float32)
                + jnp.dot(m, W1_ref[D:2 * D, :],
                          preferred_element_type=jnp.float32)
                + b1_ref[...]
            )
            hid = jnp.maximum(hid, 0.0)
            o_ref[...] = (
                jnp.dot(hid, W2_ref[...], preferred_element_type=jnp.float32)
                + b2_ref[...]
            )

    return pl.pallas_call(
        body,
        grid=grid,
        in_specs=[
            pl.BlockSpec((BLK, D), lambda i: (i, 0)),
            pl.BlockSpec((1, 1, BLK), lambda i: (i, 0, 0)),
            pl.BlockSpec((2 * D, D), lambda i: (0, 0)),
            pl.BlockSpec((1, D), lambda i: (0, 0)),
            pl.BlockSpec((D, D), lambda i: (0, 0)),
            pl.BlockSpec((1, D), lambda i: (0, 0)),
        ],
        out_specs=pl.BlockSpec((NUM_GRAPHS, D), lambda i: (0, 0)),
        out_shape=jax.ShapeDtypeStruct((NUM_GRAPHS, D), jnp.float32),
        scratch_shapes=[
            pltpu.VMEM((NUM_GRAPHS, D), jnp.float32),
            pltpu.VMEM((NUM_GRAPHS, 1), jnp.float32),
        ],
    )(h, batch3, pW1, pb1, pW2, pb2)


def kernel(x, edge_index, batch, gin_W1, gin_b1, gin_W2, gin_b2,
           post_W1, post_b1, post_W2, post_b2):
    src = edge_index[0].reshape(NW, EPT)
    dst = edge_index[1].reshape(NW, EPT)
    dummy = jnp.broadcast_to(
        N_NODES + jnp.arange(PAD_PT, dtype=jnp.int32), (NW, PAD_PT))
    src_r = jnp.concatenate(
        [src, jnp.zeros((NW, PAD_PT), jnp.int32)], axis=1
    ).reshape(NW, NIW, IW, CH)
    dst_r = jnp.concatenate(
        [dst, dummy], axis=1).reshape(NW, NIW, IW, CH)
    idx_r = jnp.stack([src_r, dst_r], axis=3).reshape(
        NW * NIW, 2 * IW, CH)

    b1 = gin_b1.reshape(1, D)
    b2 = gin_b2.reshape(1, D)

    h = jnp.pad(x, ((0, 0), (0, D - IN_FEAT)))
    for _ in range(NUM_CONVS):
        p = _sc_segment_sum(h, idx_r, D)
        h = _mlp_tc(p, h, gin_W1, b1, gin_W2, b2, D)

    batch3 = batch.reshape(N_NODES // 2000, 1, 2000)
    return _pool_post_tc(h, batch3, post_W1, post_b1.reshape(1, D),
                         post_W2, post_b2.reshape(1, D))


# speedup vs baseline: 1.0063x; 1.0063x over previous
import functools

import jax
import jax.numpy as jnp
from jax import lax
from jax.experimental import pallas as pl
from jax.experimental.pallas import tpu as pltpu
from jax.experimental.pallas import tpu_sc as plsc

N_NODES = 10000
N_EDGES = 320000
D = 128
D1 = 16
IN_FEAT = 9
NUM_GRAPHS = 16
NUM_CONVS = 3

NC = 2
NS = 16
NW = NC * NS
CH = 128
NCH = 80
IW = 80
NIW = NCH // IW
E_PAD = NW * NCH * CH
EPT = N_EDGES // NW
PAD_PT = NCH * CH - EPT
JUNK = 256
ACC_ROWS = N_NODES + JUNK
JPT = JUNK // NS

STRIPE = 624
TAIL = N_NODES - NS * STRIPE


def _sc_segment_sum(h, idx_r, d):
    mesh = plsc.VectorSubcoreMesh(core_axis_name="c", subcore_axis_name="s")

    @functools.partial(
        pl.kernel,
        mesh=mesh,
        out_type=jax.ShapeDtypeStruct((NC, N_NODES, d), jnp.float32),
        scratch_types=[
            pltpu.VMEM((2 * IW, CH), jnp.int32),
            pltpu.VMEM((CH, d), jnp.float32),
            pltpu.VMEM((CH, d), jnp.float32),
            pltpu.VMEM_SHARED((ACC_ROWS, d), jnp.float32),
            pltpu.SemaphoreType.DMA,
            pltpu.SemaphoreType.DMA,
        ],
    )
    def k(h_hbm, idx_hbm, out_hbm, idx_v, rows_a, rows_b, acc, sem_a, sem_b):
        cid = lax.axis_index("c")
        sid = lax.axis_index("s")
        wid = cid * NS + sid
        @pl.loop(0, JPT)
        def _(r):
            @pl.loop(0, d, step=16)
            def _(c):
                rows_a[r, pl.ds(c, 16)] = jnp.zeros((16,), jnp.float32)

        pltpu.sync_copy(rows_a.at[pl.ds(0, JPT)],
                        acc.at[pl.ds(N_NODES + sid * JPT, JPT)])
        r0 = sid * STRIPE
        pltpu.sync_copy(h_hbm.at[pl.ds(r0, STRIPE)],
                        acc.at[pl.ds(r0, STRIPE)])

        @pl.when(sid == NS - 1)
        def _():
            pltpu.sync_copy(h_hbm.at[pl.ds(NS * STRIPE, TAIL)],
                            acc.at[pl.ds(NS * STRIPE, TAIL)])

        plsc.subcore_barrier()

        @pl.loop(0, NIW)
        def _(w):
            pltpu.sync_copy(idx_hbm.at[wid * NIW + w], idx_v)

            @pl.loop(0, IW)
            def _(k):
                pltpu.async_copy(h_hbm.at[idx_v.at[2 * k]], rows_a,
                                 sem_a).wait()
                pltpu.sync_copy(rows_a, acc.at[idx_v.at[2 * k + 1]], add=True)

        plsc.subcore_barrier()
        pltpu.sync_copy(acc.at[pl.ds(r0, STRIPE)],
                        out_hbm.at[cid].at[pl.ds(r0, STRIPE)])

        @pl.when(sid == NS - 1)
        def _():
            pltpu.sync_copy(acc.at[pl.ds(NS * STRIPE, TAIL)],
                            out_hbm.at[cid].at[pl.ds(NS * STRIPE, TAIL)])

    return k(h, idx_r)


def _mlp_tc(p, h, W1, b1, W2, b2, d):
    BLK = 2000
    grid = (N_NODES // BLK,)

    def body(p_ref, h_ref, W1_ref, b1_ref, W2_ref, b2_ref, o_ref):
        s = p_ref[0] + p_ref[1] - h_ref[...]
        hid = jnp.dot(s, W1_ref[...], preferred_element_type=jnp.float32)
        hid = jnp.maximum(hid + b1_ref[...], 0.0)
        o_ref[...] = (
            jnp.dot(hid, W2_ref[...], preferred_element_type=jnp.float32)
            + b2_ref[...]
        )

    return pl.pallas_call(
        body,
        grid=grid,
        in_specs=[
            pl.BlockSpec((NC, BLK, d), lambda i: (0, i, 0)),
            pl.BlockSpec((BLK, d), lambda i: (i, 0)),
            pl.BlockSpec((d, D), lambda i: (0, 0)),
            pl.BlockSpec((1, D), lambda i: (0, 0)),
            pl.BlockSpec((D, D), lambda i: (0, 0)),
            pl.BlockSpec((1, D), lambda i: (0, 0)),
        ],
        out_specs=pl.BlockSpec((BLK, D), lambda i: (i, 0)),
        out_shape=jax.ShapeDtypeStruct((N_NODES, D), jnp.float32),
    )(p, h, W1[:d], b1, W2, b2)


def _pool_post_tc(h, batch3, pW1, pb1, pW2, pb2):
    BLK = 2000
    grid = (N_NODES // BLK,)

    def body(h_ref, b_ref, W1_ref, b1_ref, W2_ref, b2_ref, o_ref, sums, cnts):
        i = pl.program_id(0)

        @pl.when(i == 0)
        def _():
            sums[...] = jnp.zeros_like(sums)
            cnts[...] = jnp.zeros_like(cnts)

        gids = lax.broadcasted_iota(jnp.int32, (NUM_GRAPHS, BLK), 0)
        onehot = (gids == b_ref[0]).astype(jnp.float32)
        sums[...] += jnp.dot(onehot, h_ref[...],
                             preferred_element_type=jnp.float32)
        cnts[...] += jnp.sum(onehot, axis=1, keepdims=True)

        @pl.when(i == grid[0] - 1)
        def _():
            s = sums[...]
            m = s / jnp.maximum(cnts[...], 1.0)
            hid = (
                jnp.dot(s, W1_ref[0:D, :], preferred_element_type=jnp.float32)
                + jnp.dot(m, W1_ref[D:2 * D, :],
                          preferred_element_type=jnp.float32)
                + b1_ref[...]
            )
            hid = jnp.maximum(hid, 0.0)
            o_ref[...] = (
                jnp.dot(hid, W2_ref[...], preferred_element_type=jnp.float32)
                + b2_ref[...]
            )

    return pl.pallas_call(
        body,
        grid=grid,
        in_specs=[
            pl.BlockSpec((BLK, D), lambda i: (i, 0)),
            pl.BlockSpec((1, 1, BLK), lambda i: (i, 0, 0)),
            pl.BlockSpec((2 * D, D), lambda i: (0, 0)),
            pl.BlockSpec((1, D), lambda i: (0, 0)),
            pl.BlockSpec((D, D), lambda i: (0, 0)),
            pl.BlockSpec((1, D), lambda i: (0, 0)),
        ],
        out_specs=pl.BlockSpec((NUM_GRAPHS, D), lambda i: (0, 0)),
        out_shape=jax.ShapeDtypeStruct((NUM_GRAPHS, D), jnp.float32),
        scratch_shapes=[
            pltpu.VMEM((NUM_GRAPHS, D), jnp.float32),
            pltpu.VMEM((NUM_GRAPHS, 1), jnp.float32),
        ],
    )(h, batch3, pW1, pb1, pW2, pb2)


def kernel(x, edge_index, batch, gin_W1, gin_b1, gin_W2, gin_b2,
           post_W1, post_b1, post_W2, post_b2):
    src = edge_index[0].reshape(NW, EPT)
    dst = edge_index[1].reshape(NW, EPT)
    dummy = jnp.broadcast_to(
        N_NODES + jnp.arange(PAD_PT, dtype=jnp.int32), (NW, PAD_PT))
    src_r = jnp.concatenate(
        [src, jnp.zeros((NW, PAD_PT), jnp.int32)], axis=1
    ).reshape(NW, NIW, IW, CH)
    dst_r = jnp.concatenate(
        [dst, dummy], axis=1).reshape(NW, NIW, IW, CH)
    idx_r = jnp.stack([src_r, dst_r], axis=3).reshape(
        NW * NIW, 2 * IW, CH)

    b1 = gin_b1.reshape(1, D)
    b2 = gin_b2.reshape(1, D)

    h = jnp.pad(x, ((0, 0), (0, D - IN_FEAT)))
    for _ in range(NUM_CONVS):
        p = _sc_segment_sum(h, idx_r, D)
        h = _mlp_tc(p, h, gin_W1, b1, gin_W2, b2, D)

    batch3 = batch.reshape(N_NODES // 2000, 1, 2000)
    return _pool_post_tc(h, batch3, post_W1, post_b1.reshape(1, D),
                         post_W2, post_b2.reshape(1, D))


# speedup vs baseline: 1.4905x; 1.4812x over previous
import functools

import jax
import jax.numpy as jnp
from jax import lax
from jax.experimental import pallas as pl
from jax.experimental.pallas import tpu as pltpu
from jax.experimental.pallas import tpu_sc as plsc

N_NODES = 10000
N_EDGES = 320000
D = 128
IN_FEAT = 9
NUM_GRAPHS = 16
NUM_CONVS = 3

NC = 2
NS = 16
NW = NC * NS
CH = 128
NCH = 79
E_PAD = NW * NCH * CH
EPT = N_EDGES // NW
PAD_PT = NCH * CH - EPT
JUNK = 128
ACC_ROWS = N_NODES + JUNK
JPT = JUNK // NS

STRIPE = 624
TAIL = N_NODES - NS * STRIPE


def _sc_segment_sum(h, src_r, dst_r, d):
    mesh = plsc.VectorSubcoreMesh(core_axis_name="c", subcore_axis_name="s")

    @functools.partial(
        pl.kernel,
        mesh=mesh,
        out_type=jax.ShapeDtypeStruct((NC, N_NODES, d), jnp.float32),
        scratch_types=[
            pltpu.VMEM((NCH, CH), jnp.int32),
            pltpu.VMEM((NCH, CH), jnp.int32),
            pltpu.VMEM((CH, d), jnp.float32),
            pltpu.VMEM_SHARED((ACC_ROWS, d), jnp.float32),
            pltpu.SemaphoreType.DMA,
        ],
    )
    def k(h_hbm, src_hbm, dst_hbm, out_hbm, src_v, dst_v, rows_v, acc, sem):
        cid = lax.axis_index("c")
        sid = lax.axis_index("s")
        wid = cid * NS + sid
        pltpu.sync_copy(src_hbm.at[wid], src_v)
        pltpu.sync_copy(dst_hbm.at[wid], dst_v)
        @pl.loop(0, JPT)
        def _(r):
            @pl.loop(0, d, step=16)
            def _(c):
                rows_v[r, pl.ds(c, 16)] = jnp.zeros((16,), jnp.float32)

        pltpu.sync_copy(rows_v.at[pl.ds(0, JPT)],
                        acc.at[pl.ds(N_NODES + sid * JPT, JPT)])
        r0 = sid * STRIPE
        pltpu.sync_copy(h_hbm.at[pl.ds(r0, STRIPE)],
                        acc.at[pl.ds(r0, STRIPE)])

        @pl.when(sid == NS - 1)
        def _():
            pltpu.sync_copy(h_hbm.at[pl.ds(NS * STRIPE, TAIL)],
                            acc.at[pl.ds(NS * STRIPE, TAIL)])

        plsc.subcore_barrier()

        @pl.loop(0, NCH)
        def _(j):
            pltpu.async_copy(h_hbm.at[src_v.at[j]], rows_v, sem).wait()
            pltpu.sync_copy(rows_v, acc.at[dst_v.at[j]], add=True)

        plsc.subcore_barrier()
        pltpu.sync_copy(acc.at[pl.ds(r0, STRIPE)],
                        out_hbm.at[cid].at[pl.ds(r0, STRIPE)])

        @pl.when(sid == NS - 1)
        def _():
            pltpu.sync_copy(acc.at[pl.ds(NS * STRIPE, TAIL)],
                            out_hbm.at[cid].at[pl.ds(NS * STRIPE, TAIL)])

    return k(h, src_r, dst_r)


def _mlp_tc(p, h, W1, b1, W2, b2, d):
    BLK = 2000
    grid = (N_NODES // BLK,)

    def body(p_ref, h_ref, W1_ref, b1_ref, W2_ref, b2_ref, o_ref):
        s = p_ref[0] + p_ref[1] - h_ref[...]
        hid = jnp.dot(s, W1_ref[...], preferred_element_type=jnp.float32)
        hid = jnp.maximum(hid + b1_ref[...], 0.0)
        o_ref[...] = (
            jnp.dot(hid, W2_ref[...], preferred_element_type=jnp.float32)
            + b2_ref[...]
        )

    return pl.pallas_call(
        body,
        grid=grid,
        in_specs=[
            pl.BlockSpec((NC, BLK, d), lambda i: (0, i, 0)),
            pl.BlockSpec((BLK, d), lambda i: (i, 0)),
            pl.BlockSpec((d, D), lambda i: (0, 0)),
            pl.BlockSpec((1, D), lambda i: (0, 0)),
            pl.BlockSpec((D, D), lambda i: (0, 0)),
            pl.BlockSpec((1, D), lambda i: (0, 0)),
        ],
        out_specs=pl.BlockSpec((BLK, D), lambda i: (i, 0)),
        out_shape=jax.ShapeDtypeStruct((N_NODES, D), jnp.float32),
    )(p, h, W1[:d], b1, W2, b2)


def _pool_post_tc(h, batch3, pW1, pb1, pW2, pb2):
    BLK = 2000
    grid = (N_NODES // BLK,)

    def body(h_ref, b_ref, W1_ref, b1_ref, W2_ref, b2_ref, o_ref, sums, cnts):
        i = pl.program_id(0)

        @pl.when(i == 0)
        def _():
            sums[...] = jnp.zeros_like(sums)
            cnts[...] = jnp.zeros_like(cnts)

        gids = lax.broadcasted_iota(jnp.int32, (NUM_GRAPHS, BLK), 0)
        onehot = (gids == b_ref[0]).astype(jnp.float32)
        sums[...] += jnp.dot(onehot, h_ref[...],
                             preferred_element_type=jnp.float32)
        cnts[...] += jnp.sum(onehot, axis=1, keepdims=True)

        @pl.when(i == grid[0] - 1)
        def _():
            s = sums[...]
            m = s / jnp.maximum(cnts[...], 1.0)
            hid = (
                jnp.dot(s, W1_ref[0:D, :], preferred_element_type=jnp.float32)
                + jnp.dot(m, W1_ref[D:2 * D, :],
                          preferred_element_type=jnp.float32)
                + b1_ref[...]
            )
            hid = jnp.maximum(hid, 0.0)
            o_ref[...] = (
                jnp.dot(hid, W2_ref[...], preferred_element_type=jnp.float32)
                + b2_ref[...]
            )

    return pl.pallas_call(
        body,
        grid=grid,
        in_specs=[
            pl.BlockSpec((BLK, D), lambda i: (i, 0)),
            pl.BlockSpec((1, 1, BLK), lambda i: (i, 0, 0)),
            pl.BlockSpec((2 * D, D), lambda i: (0, 0)),
            pl.BlockSpec((1, D), lambda i: (0, 0)),
            pl.BlockSpec((D, D), lambda i: (0, 0)),
            pl.BlockSpec((1, D), lambda i: (0, 0)),
        ],
        out_specs=pl.BlockSpec((NUM_GRAPHS, D), lambda i: (0, 0)),
        out_shape=jax.ShapeDtypeStruct((NUM_GRAPHS, D), jnp.float32),
        scratch_shapes=[
            pltpu.VMEM((NUM_GRAPHS, D), jnp.float32),
            pltpu.VMEM((NUM_GRAPHS, 1), jnp.float32),
        ],
    )(h, batch3, pW1, pb1, pW2, pb2)


def kernel(x, edge_index, batch, gin_W1, gin_b1, gin_W2, gin_b2,
           post_W1, post_b1, post_W2, post_b2):
    src = edge_index[0].reshape(NW, EPT)
    dst = edge_index[1].reshape(NW, EPT)
    dummy = jnp.broadcast_to(
        N_NODES + jnp.arange(PAD_PT, dtype=jnp.int32), (NW, PAD_PT))
    src_r = jnp.concatenate(
        [src, jnp.zeros((NW, PAD_PT), jnp.int32)], axis=1
    ).reshape(NW, NCH, CH)
    dst_r = jnp.concatenate(
        [dst, dummy], axis=1).reshape(NW, NCH, CH)

    b1 = gin_b1.reshape(1, D)
    b2 = gin_b2.reshape(1, D)

    h = jnp.pad(x, ((0, 0), (0, D - IN_FEAT)))
    for _ in range(NUM_CONVS):
        p = _sc_segment_sum(h, src_r, dst_r, D)
        h = _mlp_tc(p, h, gin_W1, b1, gin_W2, b2, D)

    batch3 = batch.reshape(N_NODES // 2000, 1, 2000)
    return _pool_post_tc(h, batch3, post_W1, post_b1.reshape(1, D),
                         post_W2, post_b2.reshape(1, D))
